# baseline (speedup 1.0000x reference)
import functools
import jax
import jax.numpy as jnp
from jax.experimental import pallas as pl
from jax.experimental.pallas import tpu as pltpu

_H = 1024
_E = 8
_K = 2
_F = 4096
_T = 4096

_B = 256
_P = _T * _K + _E * _B
_NB = _P // _B
_FB = 1024
_NF = _F // _FB

_NEG = -3e38


def _gating_body(x_ref, wg_ref, bg_ref, out_ref):
    s = jnp.dot(x_ref[...], wg_ref[...], preferred_element_type=jnp.float32)
    s = s + bg_ref[...]
    col = jax.lax.broadcasted_iota(jnp.int32, s.shape, 1)
    s = jnp.where(col < _E, s, _NEG)
    m1 = jnp.max(s, axis=1, keepdims=True)
    i1 = jnp.min(jnp.where(s == m1, col, 128), axis=1, keepdims=True)
    s2 = jnp.where(col == i1, _NEG, s)
    m2 = jnp.max(s2, axis=1, keepdims=True)
    i2 = jnp.min(jnp.where(s2 == m2, col, 128), axis=1, keepdims=True)
    z = jnp.exp(m2 - m1)
    w1 = 1.0 / (1.0 + z)
    w2 = z / (1.0 + z)
    out = jnp.where(col == 0, i1.astype(jnp.float32),
          jnp.where(col == 1, i2.astype(jnp.float32),
          jnp.where(col == 2, w1,
          jnp.where(col == 3, w2, 0.0))))
    out_ref[...] = out


def _gating(x, Wg, bg):
    tb = 1024
    wgp = jnp.zeros((_H, 128), jnp.float32).at[:, :_E].set(Wg)
    bgp = jnp.zeros((1, 128), jnp.float32).at[0, :_E].set(bg)
    out = pl.pallas_call(
        _gating_body,
        grid=(_T // tb,),
        in_specs=[
            pl.BlockSpec((tb, _H), lambda i: (i, 0)),
            pl.BlockSpec((_H, 128), lambda i: (0, 0)),
            pl.BlockSpec((1, 128), lambda i: (0, 0)),
        ],
        out_specs=pl.BlockSpec((tb, 128), lambda i: (i, 0)),
        out_shape=jax.ShapeDtypeStruct((_T, 128), jnp.float32),
    )(x, wgp, bgp)
    idx = out[:, :2].astype(jnp.int32)
    w = out[:, 2:4]
    return idx, w


def _ffn1_body(meta_ref, xs_ref, w1_ref, b1_ref, h_ref):
    b = pl.program_id(0)
    nvalid = meta_ref[_NB]

    @pl.when(b < nvalid)
    def _():
        xb = xs_ref[...].astype(jnp.bfloat16)
        h = jnp.dot(xb, w1_ref[0], preferred_element_type=jnp.float32)
        h_ref[...] = jnp.maximum(h + b1_ref[0], 0.0).astype(jnp.bfloat16)


def _ffn2_body(meta_ref, h_ref, w2_ref, b2_ref, ys_ref):
    b = pl.program_id(0)
    nvalid = meta_ref[_NB]

    @pl.when(b < nvalid)
    def _():
        acc = jnp.dot(h_ref[...], w2_ref[0], preferred_element_type=jnp.float32)
        ys_ref[...] = acc + b2_ref[0]


def _grouped_ffn(meta, xs, W1, b1, W2, b2):
    gs1 = pltpu.PrefetchScalarGridSpec(
        num_scalar_prefetch=1,
        grid=(_NB,),
        in_specs=[
            pl.BlockSpec((_B, _H), lambda b, m: (b, 0)),
            pl.BlockSpec((1, _H, _F), lambda b, m: (m[b], 0, 0)),
            pl.BlockSpec((1, 1, _F), lambda b, m: (m[b], 0, 0)),
        ],
        out_specs=pl.BlockSpec((_B, _F), lambda b, m: (b, 0)),
    )
    h = pl.pallas_call(
        _ffn1_body,
        grid_spec=gs1,
        out_shape=jax.ShapeDtypeStruct((_P, _F), jnp.bfloat16),
    )(meta, xs, W1.astype(jnp.bfloat16), b1.reshape(_E, 1, _F))

    gs2 = pltpu.PrefetchScalarGridSpec(
        num_scalar_prefetch=1,
        grid=(_NB,),
        in_specs=[
            pl.BlockSpec((_B, _F), lambda b, m: (b, 0)),
            pl.BlockSpec((1, _F, _H), lambda b, m: (m[b], 0, 0)),
            pl.BlockSpec((1, 1, _H), lambda b, m: (m[b], 0, 0)),
        ],
        out_specs=pl.BlockSpec((_B, _H), lambda b, m: (b, 0)),
    )
    return pl.pallas_call(
        _ffn2_body,
        grid_spec=gs2,
        out_shape=jax.ShapeDtypeStruct((_P, _H), jnp.float32),
    )(meta, h, W2.astype(jnp.bfloat16), b2.reshape(_E, 1, _H))


@jax.jit
def kernel(x, Wg, bg, W1, b1, W2, b2):
    idx, w = _gating(x, Wg, bg)

    flat_e = idx.reshape(-1)
    counts = jnp.sum(flat_e[:, None] == jnp.arange(_E)[None, :], axis=0)
    padded = ((counts + _B - 1) // _B) * _B
    starts = jnp.concatenate([jnp.zeros((1,), jnp.int32),
                              jnp.cumsum(padded).astype(jnp.int32)])
    cexcl = jnp.concatenate([jnp.zeros((1,), jnp.int32),
                             jnp.cumsum(counts).astype(jnp.int32)])[:_E]
    order = jnp.argsort(flat_e, stable=True).astype(jnp.int32)
    seg = flat_e[order]
    dest_sorted = starts[seg] + (jnp.arange(_T * _K, dtype=jnp.int32) - cexcl[seg])
    dest = jnp.zeros((_T * _K,), jnp.int32).at[order].set(dest_sorted)
    src = jnp.zeros((_P,), jnp.int32).at[dest_sorted].set(order // _K)
    nvalid = starts[_E] // _B
    block_expert = jnp.minimum(
        jnp.searchsorted(starts[1:], jnp.arange(_NB, dtype=jnp.int32) * _B,
                         side='right'), _E - 1).astype(jnp.int32)
    meta = jnp.concatenate([block_expert, nvalid[None].astype(jnp.int32)])

    xs = x[src]
    ys = _grouped_ffn(meta, xs, W1, b1, W2, b2)

    wflat = w.reshape(-1)
    gathered = ys[dest]
    out = (gathered * wflat[:, None]).reshape(_T, _K, _H).sum(axis=1)
    return out


# speedup vs baseline: 1.4242x; 1.4242x over previous
import functools
import jax
import jax.numpy as jnp
from jax import lax
from jax.experimental import pallas as pl
from jax.experimental.pallas import tpu as pltpu
from jax.experimental.pallas import tpu_sc as plsc

_H = 1024
_E = 8
_K = 2
_F = 4096
_T = 4096
_A = _T * _K

_B = 256
_P = _A + _E * _B
_NB = _P // _B
_NM = 48

_NC = 2
_NS = 16
_NW = _NC * _NS
_CA = _A // _NW
_CT = _T // _NW

_NEG = -3e38


def _gating_body(x_ref, wg_ref, bg_ref, out_ref):
    s = jnp.dot(x_ref[...], wg_ref[...], preferred_element_type=jnp.float32)
    s = s + bg_ref[...]
    col = jax.lax.broadcasted_iota(jnp.int32, s.shape, 1)
    s = jnp.where(col < _E, s, _NEG)
    m1 = jnp.max(s, axis=1, keepdims=True)
    i1 = jnp.min(jnp.where(s == m1, col, 128), axis=1, keepdims=True)
    s2 = jnp.where(col == i1, _NEG, s)
    m2 = jnp.max(s2, axis=1, keepdims=True)
    i2 = jnp.min(jnp.where(s2 == m2, col, 128), axis=1, keepdims=True)
    z = jnp.exp(m2 - m1)
    w1 = 1.0 / (1.0 + z)
    w2 = z / (1.0 + z)
    out = jnp.where(col == 0, i1.astype(jnp.float32),
          jnp.where(col == 1, i2.astype(jnp.float32),
          jnp.where(col == 2, w1,
          jnp.where(col == 3, w2, 0.0))))
    out_ref[...] = out


def _gating(x, Wg, bg):
    tb = 1024
    wgp = jnp.zeros((_H, 128), jnp.float32).at[:, :_E].set(Wg)
    bgp = jnp.zeros((1, 128), jnp.float32).at[0, :_E].set(bg)
    out = pl.pallas_call(
        _gating_body,
        grid=(_T // tb,),
        in_specs=[
            pl.BlockSpec((tb, _H), lambda i: (i, 0)),
            pl.BlockSpec((_H, 128), lambda i: (0, 0)),
            pl.BlockSpec((1, 128), lambda i: (0, 0)),
        ],
        out_specs=pl.BlockSpec((tb, 128), lambda i: (i, 0)),
        out_shape=jax.ShapeDtypeStruct((_T, 128), jnp.float32),
    )(x, wgp, bgp)
    idx = out[:, :2].astype(jnp.int32)
    w = out[:, 2:4]
    return idx, w


_MESH = plsc.VectorSubcoreMesh(core_axis_name="c", subcore_axis_name="s")


def _wid():
    return lax.axis_index("s") * _NC + lax.axis_index("c")


@functools.partial(
    pl.kernel, mesh=_MESH,
    out_type=jax.ShapeDtypeStruct((_NW * 16,), jnp.int32),
    scratch_types=[
        pltpu.VMEM((_CA,), jnp.int32),
        pltpu.VMEM((16,), jnp.int32),
    ],
    compiler_params=pltpu.CompilerParams(needs_layout_passes=False),
)
def _count_kernel(e_hbm, counts_hbm, e_v, cnt_v):
    w = _wid()
    pltpu.sync_copy(e_hbm.at[pl.ds(w * _CA, _CA)], e_v)
    lanes = lax.iota(jnp.int32, 16)
    acc = jnp.zeros((16,), jnp.int32)
    for k in range(_CA // 16):
        v = e_v[pl.ds(k * 16, 16)]
        for e in range(_E):
            c = plsc.all_reduce_population_count(v == e)
            acc = acc + jnp.where(lanes == e, c, 0)
    cnt_v[...] = acc
    pltpu.sync_copy(cnt_v, counts_hbm.at[pl.ds(w * 16, 16)])


@functools.partial(
    pl.kernel, mesh=_MESH,
    out_type=(
        jax.ShapeDtypeStruct((_P, _H), jnp.float32),
        jax.ShapeDtypeStruct((_A,), jnp.int32),
        jax.ShapeDtypeStruct((_NM,), jnp.int32),
    ),
    scratch_types=[
        pltpu.VMEM((_NW * 16,), jnp.int32),
        pltpu.VMEM((_CA,), jnp.int32),
        pltpu.VMEM((_CA,), jnp.int32),
        pltpu.VMEM((32, _H), jnp.float32),
        pltpu.VMEM((32,), jnp.int32),
        pltpu.VMEM((32,), jnp.int32),
        pltpu.VMEM((_NM,), jnp.int32),
        pltpu.SemaphoreType.DMA,
    ],
    compiler_params=pltpu.CompilerParams(needs_layout_passes=False),
)
def _dispatch_kernel(e_hbm, x_hbm, counts_hbm, xs_hbm, dest_hbm, meta_hbm,
                     cnts_v, e_v, dest_v, rows_v, idx0_v, idx1_v, meta_v, sem):
    w = _wid()
    lanes = lax.iota(jnp.int32, 16)
    wsplat = jnp.zeros((16,), jnp.int32) + w

    pltpu.sync_copy(counts_hbm, cnts_v)
    carry = jnp.zeros((16,), jnp.int32)
    my_carry = jnp.zeros((16,), jnp.int32)
    for wp in range(_NW):
        cw = cnts_v[pl.ds(wp * 16, 16)]
        my_carry = jnp.where(wsplat == wp, carry, my_carry)
        carry = carry + cw
    total = carry
    padded = ((total + (_B - 1)) // _B) * _B
    incl = plsc.cumsum(padded)
    starts = incl - padded
    base = starts + my_carry

    base_e = []
    for e in range(_E):
        be = jnp.sum(jnp.where(lanes == e, base, 0))
        base_e.append(jnp.zeros((16,), jnp.int32) + be)

    pltpu.sync_copy(e_hbm.at[pl.ds(w * _CA, _CA)], e_v)
    for k in range(_CA // 16):
        v = e_v[pl.ds(k * 16, 16)]
        d = jnp.zeros((16,), jnp.int32)
        for e in range(_E):
            m = v == e
            pc = plsc.cumsum(jnp.where(m, 1, 0))
            d = jnp.where(m, base_e[e] + pc - 1, d)
            c = plsc.all_reduce_population_count(m)
            base_e[e] = base_e[e] + c
        dest_v[pl.ds(k * 16, 16)] = d
    pltpu.sync_copy(dest_v, dest_hbm.at[pl.ds(w * _CA, _CA)])

    for sub in range(_CT // 32):
        tok0 = w * _CT + sub * 32
        pltpu.sync_copy(x_hbm.at[pl.ds(tok0, 32)], rows_v)
        for g in range(2):
            ii = lax.iota(jnp.int32, 16) * 2 + (sub * 64 + g * 32)
            idx0_v[pl.ds(g * 16, 16)] = plsc.load_gather(dest_v, [ii])
            idx1_v[pl.ds(g * 16, 16)] = plsc.load_gather(dest_v, [ii + 1])
        pltpu.async_copy(rows_v, xs_hbm.at[idx0_v], sem).wait()
        pltpu.async_copy(rows_v, xs_hbm.at[idx1_v], sem).wait()

    @pl.when(w == 0)
    def _():
        endblk = incl // _B
        nvalid = jnp.zeros((16,), jnp.int32) + (jnp.sum(padded) // _B)
        for r in range(_NM // 16):
            bvec = lax.iota(jnp.int32, 16) + r * 16
            bexp = jnp.zeros((16,), jnp.int32)
            for e in range(_E):
                ee = jnp.zeros((16,), jnp.int32) + jnp.sum(
                    jnp.where(lanes == e, endblk, 0))
                bexp = bexp + jnp.where(bvec >= ee, 1, 0)
            mval = jnp.minimum(bexp, _E - 1)
            mval = jnp.where(bvec == _NB, nvalid, mval)
            meta_v[pl.ds(r * 16, 16)] = mval
        pltpu.sync_copy(meta_v, meta_hbm)


def _ffn1_body(meta_ref, xs_ref, w1_ref, b1_ref, h_ref):
    b = pl.program_id(0)
    nvalid = meta_ref[_NB]

    @pl.when(b < nvalid)
    def _():
        h = jnp.dot(xs_ref[...], w1_ref[0], preferred_element_type=jnp.float32)
        h_ref[...] = jnp.maximum(h + b1_ref[0], 0.0)


def _ffn2_body(meta_ref, h_ref, w2_ref, b2_ref, ys_ref):
    b = pl.program_id(0)
    nvalid = meta_ref[_NB]

    @pl.when(b < nvalid)
    def _():
        acc = jnp.dot(h_ref[...], w2_ref[0], preferred_element_type=jnp.float32)
        ys_ref[...] = acc + b2_ref[0]


def _grouped_ffn(meta, xs, W1, b1, W2, b2):
    gs1 = pltpu.PrefetchScalarGridSpec(
        num_scalar_prefetch=1,
        grid=(_NB,),
        in_specs=[
            pl.BlockSpec((_B, _H), lambda b, m: (b, 0)),
            pl.BlockSpec((1, _H, _F), lambda b, m: (m[b], 0, 0)),
            pl.BlockSpec((1, 1, _F), lambda b, m: (m[b], 0, 0)),
        ],
        out_specs=pl.BlockSpec((_B, _F), lambda b, m: (b, 0)),
    )
    h = pl.pallas_call(
        _ffn1_body,
        grid_spec=gs1,
        out_shape=jax.ShapeDtypeStruct((_P, _F), jnp.float32),
    )(meta, xs, W1, b1.reshape(_E, 1, _F))

    gs2 = pltpu.PrefetchScalarGridSpec(
        num_scalar_prefetch=1,
        grid=(_NB,),
        in_specs=[
            pl.BlockSpec((_B, _F), lambda b, m: (b, 0)),
            pl.BlockSpec((1, _F, _H), lambda b, m: (m[b], 0, 0)),
            pl.BlockSpec((1, 1, _H), lambda b, m: (m[b], 0, 0)),
        ],
        out_specs=pl.BlockSpec((_B, _H), lambda b, m: (b, 0)),
    )
    return pl.pallas_call(
        _ffn2_body,
        grid_spec=gs2,
        out_shape=jax.ShapeDtypeStruct((_P, _H), jnp.float32),
    )(meta, h, W2, b2.reshape(_E, 1, _H))


@jax.jit
def kernel(x, Wg, bg, W1, b1, W2, b2):
    idx, w = _gating(x, Wg, bg)
    eflat = idx.reshape(-1)

    counts = _count_kernel(eflat)
    if False:
        cnt = counts.reshape(_NW, 16)[:, :_E].sum(axis=0)
        padded = ((cnt + _B - 1) // _B) * _B
        starts = jnp.concatenate([jnp.zeros((1,), jnp.int32),
                                  jnp.cumsum(padded).astype(jnp.int32)])
        cexcl = jnp.concatenate([jnp.zeros((1,), jnp.int32),
                                 jnp.cumsum(cnt).astype(jnp.int32)])[:_E]
        order = jnp.argsort(eflat, stable=True).astype(jnp.int32)
        seg = eflat[order]
        dest_sorted = starts[seg] + (jnp.arange(_A, dtype=jnp.int32) - cexcl[seg])
        dest = jnp.zeros((_A,), jnp.int32).at[order].set(dest_sorted)
        src = jnp.zeros((_P,), jnp.int32).at[dest_sorted].set(order // _K)
        nvalid = starts[_E] // _B
        block_expert = jnp.minimum(
            jnp.searchsorted(starts[1:], jnp.arange(_NB, dtype=jnp.int32) * _B,
                             side='right'), _E - 1).astype(jnp.int32)
        meta = jnp.concatenate([block_expert, nvalid[None].astype(jnp.int32)])
        xs = x[src]
    else:
        xs, dest, meta = _dispatch_kernel(eflat, x, counts)

    ys = _grouped_ffn(meta, xs, W1, b1, W2, b2)

    wflat = w.reshape(-1)
    gathered = ys[dest]
    out = (gathered * wflat[:, None]).reshape(_T, _K, _H).sum(axis=1)
    return out


# speedup vs baseline: 1.5083x; 1.0590x over previous
import functools
import jax
import jax.numpy as jnp
from jax import lax
from jax.experimental import pallas as pl
from jax.experimental.pallas import tpu as pltpu
from jax.experimental.pallas import tpu_sc as plsc

_H = 1024
_E = 8
_K = 2
_F = 4096
_T = 4096
_A = _T * _K

_B = 256
_P = _A + _E * _B
_NB = _P // _B
_NM = 48

_NC = 2
_NS = 16
_NW = _NC * _NS
_CA = _A // _NW
_CT = _T // _NW

_NEG = -3e38


def _gating_body(x_ref, wg_ref, bg_ref, out_ref):
    s = jnp.dot(x_ref[...], wg_ref[...], preferred_element_type=jnp.float32)
    s = s + bg_ref[...]
    col = jax.lax.broadcasted_iota(jnp.int32, s.shape, 1)
    s = jnp.where(col < _E, s, _NEG)
    m1 = jnp.max(s, axis=1, keepdims=True)
    i1 = jnp.min(jnp.where(s == m1, col, 128), axis=1, keepdims=True)
    s2 = jnp.where(col == i1, _NEG, s)
    m2 = jnp.max(s2, axis=1, keepdims=True)
    i2 = jnp.min(jnp.where(s2 == m2, col, 128), axis=1, keepdims=True)
    z = jnp.exp(m2 - m1)
    w1 = 1.0 / (1.0 + z)
    w2 = z / (1.0 + z)
    out = jnp.where(col == 0, i1.astype(jnp.float32),
          jnp.where(col == 1, i2.astype(jnp.float32),
          jnp.where(col == 2, w1,
          jnp.where(col == 3, w2, 0.0))))
    out_ref[...] = out


def _gating(x, Wg, bg):
    tb = 1024
    wgp = jnp.zeros((_H, 128), jnp.float32).at[:, :_E].set(Wg)
    bgp = jnp.zeros((1, 128), jnp.float32).at[0, :_E].set(bg)
    out = pl.pallas_call(
        _gating_body,
        grid=(_T // tb,),
        in_specs=[
            pl.BlockSpec((tb, _H), lambda i: (i, 0)),
            pl.BlockSpec((_H, 128), lambda i: (0, 0)),
            pl.BlockSpec((1, 128), lambda i: (0, 0)),
        ],
        out_specs=pl.BlockSpec((tb, 128), lambda i: (i, 0)),
        out_shape=jax.ShapeDtypeStruct((_T, 128), jnp.float32),
    )(x, wgp, bgp)
    idx = out[:, :2].astype(jnp.int32)
    w = out[:, 2:4]
    return idx, w


_MESH = plsc.VectorSubcoreMesh(core_axis_name="c", subcore_axis_name="s")


def _wid():
    return lax.axis_index("s") * _NC + lax.axis_index("c")


@functools.partial(
    pl.kernel, mesh=_MESH,
    out_type=jax.ShapeDtypeStruct((_NW * 16,), jnp.int32),
    scratch_types=[
        pltpu.VMEM((_CA,), jnp.int32),
        pltpu.VMEM((16,), jnp.int32),
    ],
    compiler_params=pltpu.CompilerParams(needs_layout_passes=False),
)
def _count_kernel(e_hbm, counts_hbm, e_v, cnt_v):
    w = _wid()
    pltpu.sync_copy(e_hbm.at[pl.ds(w * _CA, _CA)], e_v)
    lanes = lax.iota(jnp.int32, 16)
    acc = jnp.zeros((16,), jnp.int32)
    for k in range(_CA // 16):
        v = e_v[pl.ds(k * 16, 16)]
        for e in range(_E):
            c = plsc.all_reduce_population_count(v == e)
            acc = acc + jnp.where(lanes == e, c, 0)
    cnt_v[...] = acc
    pltpu.sync_copy(cnt_v, counts_hbm.at[pl.ds(w * 16, 16)])


@functools.partial(
    pl.kernel, mesh=_MESH,
    out_type=(
        jax.ShapeDtypeStruct((_P, _H), jnp.float32),
        jax.ShapeDtypeStruct((_A,), jnp.int32),
        jax.ShapeDtypeStruct((_NM,), jnp.int32),
    ),
    scratch_types=[
        pltpu.VMEM((_NW * 16,), jnp.int32),
        pltpu.VMEM((_CA,), jnp.int32),
        pltpu.VMEM((_CA,), jnp.int32),
        pltpu.VMEM((32, _H), jnp.float32),
        pltpu.VMEM((32,), jnp.int32),
        pltpu.VMEM((32,), jnp.int32),
        pltpu.VMEM((_NM,), jnp.int32),
        pltpu.SemaphoreType.DMA,
    ],
    compiler_params=pltpu.CompilerParams(needs_layout_passes=False),
)
def _dispatch_kernel(e_hbm, x_hbm, counts_hbm, xs_hbm, dest_hbm, meta_hbm,
                     cnts_v, e_v, dest_v, rows_v, idx0_v, idx1_v, meta_v, sem):
    w = _wid()
    lanes = lax.iota(jnp.int32, 16)
    wsplat = jnp.zeros((16,), jnp.int32) + w

    pltpu.sync_copy(counts_hbm, cnts_v)
    carry = jnp.zeros((16,), jnp.int32)
    my_carry = jnp.zeros((16,), jnp.int32)
    for wp in range(_NW):
        cw = cnts_v[pl.ds(wp * 16, 16)]
        my_carry = jnp.where(wsplat == wp, carry, my_carry)
        carry = carry + cw
    total = carry
    padded = ((total + (_B - 1)) // _B) * _B
    incl = plsc.cumsum(padded)
    starts = incl - padded
    base = starts + my_carry

    base_e = []
    for e in range(_E):
        be = jnp.sum(jnp.where(lanes == e, base, 0))
        base_e.append(jnp.zeros((16,), jnp.int32) + be)

    pltpu.sync_copy(e_hbm.at[pl.ds(w * _CA, _CA)], e_v)
    for k in range(_CA // 16):
        v = e_v[pl.ds(k * 16, 16)]
        d = jnp.zeros((16,), jnp.int32)
        for e in range(_E):
            m = v == e
            pc = plsc.cumsum(jnp.where(m, 1, 0))
            d = jnp.where(m, base_e[e] + pc - 1, d)
            c = plsc.all_reduce_population_count(m)
            base_e[e] = base_e[e] + c
        dest_v[pl.ds(k * 16, 16)] = d
    pltpu.sync_copy(dest_v, dest_hbm.at[pl.ds(w * _CA, _CA)])

    for sub in range(_CT // 32):
        tok0 = w * _CT + sub * 32
        pltpu.sync_copy(x_hbm.at[pl.ds(tok0, 32)], rows_v)
        for g in range(2):
            ii = lax.iota(jnp.int32, 16) * 2 + (sub * 64 + g * 32)
            idx0_v[pl.ds(g * 16, 16)] = plsc.load_gather(dest_v, [ii])
            idx1_v[pl.ds(g * 16, 16)] = plsc.load_gather(dest_v, [ii + 1])
        pltpu.async_copy(rows_v, xs_hbm.at[idx0_v], sem).wait()
        pltpu.async_copy(rows_v, xs_hbm.at[idx1_v], sem).wait()

    @pl.when(w == 0)
    def _():
        endblk = incl // _B
        nvalid = jnp.zeros((16,), jnp.int32) + (jnp.sum(padded) // _B)
        for r in range(_NM // 16):
            bvec = lax.iota(jnp.int32, 16) + r * 16
            bexp = jnp.zeros((16,), jnp.int32)
            for e in range(_E):
                ee = jnp.zeros((16,), jnp.int32) + jnp.sum(
                    jnp.where(lanes == e, endblk, 0))
                bexp = bexp + jnp.where(bvec >= ee, 1, 0)
            mval = jnp.minimum(bexp, _E - 1)
            mval = jnp.where(bvec == _NB, nvalid, mval)
            meta_v[pl.ds(r * 16, 16)] = mval
        pltpu.sync_copy(meta_v, meta_hbm)


_FH = _F // 2


def _ffn_h1_body(meta_ref, xs_ref, w1_ref, b1_ref, w2_ref, b2_ref,
                 ys_ref, h_ref):
    b = pl.program_id(0)
    nvalid = meta_ref[_NB]

    @pl.when(b < nvalid)
    def _():
        h = jnp.dot(xs_ref[...], w1_ref[0], preferred_element_type=jnp.float32)
        h_ref[...] = jnp.maximum(h + b1_ref[0], 0.0)
        acc = jnp.dot(h_ref[...], w2_ref[0], preferred_element_type=jnp.float32)
        ys_ref[...] = acc + b2_ref[0]


def _ffn_h2_body(meta_ref, xs_ref, w1_ref, b1_ref, w2_ref, prev_ref,
                 ys_ref, h_ref):
    b = pl.program_id(0)
    nvalid = meta_ref[_NB]

    @pl.when(b < nvalid)
    def _():
        h = jnp.dot(xs_ref[...], w1_ref[0], preferred_element_type=jnp.float32)
        h_ref[...] = jnp.maximum(h + b1_ref[0], 0.0)
        acc = jnp.dot(h_ref[...], w2_ref[0], preferred_element_type=jnp.float32)
        ys_ref[...] = acc + prev_ref[...]


def _grouped_ffn(meta, xs, W1, b1, W2, b2):
    gs1 = pltpu.PrefetchScalarGridSpec(
        num_scalar_prefetch=1,
        grid=(_NB,),
        in_specs=[
            pl.BlockSpec((_B, _H), lambda b, m: (b, 0)),
            pl.BlockSpec((1, _H, _FH), lambda b, m: (m[b], 0, 0)),
            pl.BlockSpec((1, 1, _FH), lambda b, m: (m[b], 0, 0)),
            pl.BlockSpec((1, _FH, _H), lambda b, m: (m[b], 0, 0)),
            pl.BlockSpec((1, 1, _H), lambda b, m: (m[b], 0, 0)),
        ],
        out_specs=pl.BlockSpec((_B, _H), lambda b, m: (b, 0)),
        scratch_shapes=[pltpu.VMEM((_B, _FH), jnp.float32)],
    )
    ys1 = pl.pallas_call(
        _ffn_h1_body,
        grid_spec=gs1,
        out_shape=jax.ShapeDtypeStruct((_P, _H), jnp.float32),
    )(meta, xs, W1, b1.reshape(_E, 1, _F), W2, b2.reshape(_E, 1, _H))

    gs2 = pltpu.PrefetchScalarGridSpec(
        num_scalar_prefetch=1,
        grid=(_NB,),
        in_specs=[
            pl.BlockSpec((_B, _H), lambda b, m: (b, 0)),
            pl.BlockSpec((1, _H, _FH), lambda b, m: (m[b], 0, 1)),
            pl.BlockSpec((1, 1, _FH), lambda b, m: (m[b], 0, 1)),
            pl.BlockSpec((1, _FH, _H), lambda b, m: (m[b], 1, 0)),
            pl.BlockSpec((_B, _H), lambda b, m: (b, 0)),
        ],
        out_specs=pl.BlockSpec((_B, _H), lambda b, m: (b, 0)),
        scratch_shapes=[pltpu.VMEM((_B, _FH), jnp.float32)],
    )
    return pl.pallas_call(
        _ffn_h2_body,
        grid_spec=gs2,
        out_shape=jax.ShapeDtypeStruct((_P, _H), jnp.float32),
    )(meta, xs, W1, b1.reshape(_E, 1, _F), W2, ys1)


@jax.jit
def kernel(x, Wg, bg, W1, b1, W2, b2):
    idx, w = _gating(x, Wg, bg)
    eflat = idx.reshape(-1)

    counts = _count_kernel(eflat)
    if False:
        cnt = counts.reshape(_NW, 16)[:, :_E].sum(axis=0)
        padded = ((cnt + _B - 1) // _B) * _B
        starts = jnp.concatenate([jnp.zeros((1,), jnp.int32),
                                  jnp.cumsum(padded).astype(jnp.int32)])
        cexcl = jnp.concatenate([jnp.zeros((1,), jnp.int32),
                                 jnp.cumsum(cnt).astype(jnp.int32)])[:_E]
        order = jnp.argsort(eflat, stable=True).astype(jnp.int32)
        seg = eflat[order]
        dest_sorted = starts[seg] + (jnp.arange(_A, dtype=jnp.int32) - cexcl[seg])
        dest = jnp.zeros((_A,), jnp.int32).at[order].set(dest_sorted)
        src = jnp.zeros((_P,), jnp.int32).at[dest_sorted].set(order // _K)
        nvalid = starts[_E] // _B
        block_expert = jnp.minimum(
            jnp.searchsorted(starts[1:], jnp.arange(_NB, dtype=jnp.int32) * _B,
                             side='right'), _E - 1).astype(jnp.int32)
        meta = jnp.concatenate([block_expert, nvalid[None].astype(jnp.int32)])
        xs = x[src]
    else:
        xs, dest, meta = _dispatch_kernel(eflat, x, counts)

    ys = _grouped_ffn(meta, xs, W1, b1, W2, b2)

    wflat = w.reshape(-1)
    gathered = ys[dest]
    out = (gathered * wflat[:, None]).reshape(_T, _K, _H).sum(axis=1)
    return out


# speedup vs baseline: 1.5379x; 1.0197x over previous
import functools
import jax
import jax.numpy as jnp
from jax import lax
from jax.experimental import pallas as pl
from jax.experimental.pallas import tpu as pltpu
from jax.experimental.pallas import tpu_sc as plsc

_H = 1024
_E = 8
_K = 2
_F = 4096
_T = 4096
_A = _T * _K

_B = 512
_P = _A + _E * _B
_NB = _P // _B
_NM = 48

_NC = 2
_NS = 16
_NW = _NC * _NS
_CA = _A // _NW
_CT = _T // _NW

_NEG = -3e38


def _gating_body(x_ref, wg_ref, bg_ref, out_ref):
    s = jnp.dot(x_ref[...], wg_ref[...], preferred_element_type=jnp.float32)
    s = s + bg_ref[...]
    col = jax.lax.broadcasted_iota(jnp.int32, s.shape, 1)
    s = jnp.where(col < _E, s, _NEG)
    m1 = jnp.max(s, axis=1, keepdims=True)
    i1 = jnp.min(jnp.where(s == m1, col, 128), axis=1, keepdims=True)
    s2 = jnp.where(col == i1, _NEG, s)
    m2 = jnp.max(s2, axis=1, keepdims=True)
    i2 = jnp.min(jnp.where(s2 == m2, col, 128), axis=1, keepdims=True)
    z = jnp.exp(m2 - m1)
    w1 = 1.0 / (1.0 + z)
    w2 = z / (1.0 + z)
    out = jnp.where(col == 0, i1.astype(jnp.float32),
          jnp.where(col == 1, i2.astype(jnp.float32),
          jnp.where(col == 2, w1,
          jnp.where(col == 3, w2, 0.0))))
    out_ref[...] = out


def _gating(x, Wg, bg):
    tb = 1024
    wgp = jnp.zeros((_H, 128), jnp.float32).at[:, :_E].set(Wg)
    bgp = jnp.zeros((1, 128), jnp.float32).at[0, :_E].set(bg)
    out = pl.pallas_call(
        _gating_body,
        grid=(_T // tb,),
        in_specs=[
            pl.BlockSpec((tb, _H), lambda i: (i, 0)),
            pl.BlockSpec((_H, 128), lambda i: (0, 0)),
            pl.BlockSpec((1, 128), lambda i: (0, 0)),
        ],
        out_specs=pl.BlockSpec((tb, 128), lambda i: (i, 0)),
        out_shape=jax.ShapeDtypeStruct((_T, 128), jnp.float32),
    )(x, wgp, bgp)
    idx = out[:, :2].astype(jnp.int32)
    w = out[:, 2:4]
    return idx, w


_MESH = plsc.VectorSubcoreMesh(core_axis_name="c", subcore_axis_name="s")


def _wid():
    return lax.axis_index("s") * _NC + lax.axis_index("c")


@functools.partial(
    pl.kernel, mesh=_MESH,
    out_type=jax.ShapeDtypeStruct((_NW * 16,), jnp.int32),
    scratch_types=[
        pltpu.VMEM((_CA,), jnp.int32),
        pltpu.VMEM((16,), jnp.int32),
    ],
    compiler_params=pltpu.CompilerParams(needs_layout_passes=False),
)
def _count_kernel(e_hbm, counts_hbm, e_v, cnt_v):
    w = _wid()
    pltpu.sync_copy(e_hbm.at[pl.ds(w * _CA, _CA)], e_v)
    lanes = lax.iota(jnp.int32, 16)
    acc = jnp.zeros((16,), jnp.int32)
    for k in range(_CA // 16):
        v = e_v[pl.ds(k * 16, 16)]
        for e in range(_E):
            c = plsc.all_reduce_population_count(v == e)
            acc = acc + jnp.where(lanes == e, c, 0)
    cnt_v[...] = acc
    pltpu.sync_copy(cnt_v, counts_hbm.at[pl.ds(w * 16, 16)])


@functools.partial(
    pl.kernel, mesh=_MESH,
    out_type=(
        jax.ShapeDtypeStruct((_P, _H), jnp.float32),
        jax.ShapeDtypeStruct((_A,), jnp.int32),
        jax.ShapeDtypeStruct((_NM,), jnp.int32),
    ),
    scratch_types=[
        pltpu.VMEM((_NW * 16,), jnp.int32),
        pltpu.VMEM((_CA,), jnp.int32),
        pltpu.VMEM((_CA,), jnp.int32),
        pltpu.VMEM((32, _H), jnp.float32),
        pltpu.VMEM((32,), jnp.int32),
        pltpu.VMEM((32,), jnp.int32),
        pltpu.VMEM((_NM,), jnp.int32),
        pltpu.SemaphoreType.DMA,
    ],
    compiler_params=pltpu.CompilerParams(needs_layout_passes=False),
)
def _dispatch_kernel(e_hbm, x_hbm, counts_hbm, xs_hbm, dest_hbm, meta_hbm,
                     cnts_v, e_v, dest_v, rows_v, idx0_v, idx1_v, meta_v, sem):
    w = _wid()
    lanes = lax.iota(jnp.int32, 16)
    wsplat = jnp.zeros((16,), jnp.int32) + w

    pltpu.sync_copy(counts_hbm, cnts_v)
    carry = jnp.zeros((16,), jnp.int32)
    my_carry = jnp.zeros((16,), jnp.int32)
    for wp in range(_NW):
        cw = cnts_v[pl.ds(wp * 16, 16)]
        my_carry = jnp.where(wsplat == wp, carry, my_carry)
        carry = carry + cw
    total = carry
    padded = ((total + (_B - 1)) // _B) * _B
    incl = plsc.cumsum(padded)
    starts = incl - padded
    base = starts + my_carry

    base_e = []
    for e in range(_E):
        be = jnp.sum(jnp.where(lanes == e, base, 0))
        base_e.append(jnp.zeros((16,), jnp.int32) + be)

    pltpu.sync_copy(e_hbm.at[pl.ds(w * _CA, _CA)], e_v)
    for k in range(_CA // 16):
        v = e_v[pl.ds(k * 16, 16)]
        d = jnp.zeros((16,), jnp.int32)
        for e in range(_E):
            m = v == e
            pc = plsc.cumsum(jnp.where(m, 1, 0))
            d = jnp.where(m, base_e[e] + pc - 1, d)
            c = plsc.all_reduce_population_count(m)
            base_e[e] = base_e[e] + c
        dest_v[pl.ds(k * 16, 16)] = d
    pltpu.sync_copy(dest_v, dest_hbm.at[pl.ds(w * _CA, _CA)])

    for sub in range(_CT // 32):
        tok0 = w * _CT + sub * 32
        pltpu.sync_copy(x_hbm.at[pl.ds(tok0, 32)], rows_v)
        for g in range(2):
            ii = lax.iota(jnp.int32, 16) * 2 + (sub * 64 + g * 32)
            idx0_v[pl.ds(g * 16, 16)] = plsc.load_gather(dest_v, [ii])
            idx1_v[pl.ds(g * 16, 16)] = plsc.load_gather(dest_v, [ii + 1])
        pltpu.async_copy(rows_v, xs_hbm.at[idx0_v], sem).wait()
        pltpu.async_copy(rows_v, xs_hbm.at[idx1_v], sem).wait()

    @pl.when(w == 0)
    def _():
        endblk = incl // _B
        nvalid = jnp.zeros((16,), jnp.int32) + (jnp.sum(padded) // _B)
        for r in range(_NM // 16):
            bvec = lax.iota(jnp.int32, 16) + r * 16
            bexp = jnp.zeros((16,), jnp.int32)
            for e in range(_E):
                ee = jnp.zeros((16,), jnp.int32) + jnp.sum(
                    jnp.where(lanes == e, endblk, 0))
                bexp = bexp + jnp.where(bvec >= ee, 1, 0)
            mval = jnp.minimum(bexp, _E - 1)
            mval = jnp.where(bvec == _NB, nvalid, mval)
            meta_v[pl.ds(r * 16, 16)] = mval
        pltpu.sync_copy(meta_v, meta_hbm)


_FH = _F // 2


def _ffn_h1_body(meta_ref, xs_ref, w1_ref, b1_ref, w2_ref, b2_ref,
                 ys_ref, h_ref):
    b = pl.program_id(0)
    nvalid = meta_ref[_NB]

    @pl.when(b < nvalid)
    def _():
        h = jnp.dot(xs_ref[...], w1_ref[0], preferred_element_type=jnp.float32)
        h_ref[...] = jnp.maximum(h + b1_ref[0], 0.0)
        acc = jnp.dot(h_ref[...], w2_ref[0], preferred_element_type=jnp.float32)
        ys_ref[...] = acc + b2_ref[0]


def _ffn_h2_body(meta_ref, xs_ref, w1_ref, b1_ref, w2_ref, prev_ref,
                 ys_ref, h_ref):
    b = pl.program_id(0)
    nvalid = meta_ref[_NB]

    @pl.when(b < nvalid)
    def _():
        h = jnp.dot(xs_ref[...], w1_ref[0], preferred_element_type=jnp.float32)
        h_ref[...] = jnp.maximum(h + b1_ref[0], 0.0)
        acc = jnp.dot(h_ref[...], w2_ref[0], preferred_element_type=jnp.float32)
        ys_ref[...] = acc + prev_ref[...]


def _grouped_ffn(meta, xs, W1, b1, W2, b2):
    gs1 = pltpu.PrefetchScalarGridSpec(
        num_scalar_prefetch=1,
        grid=(_NB,),
        in_specs=[
            pl.BlockSpec((_B, _H), lambda b, m: (b, 0)),
            pl.BlockSpec((1, _H, _FH), lambda b, m: (m[b], 0, 0)),
            pl.BlockSpec((1, 1, _FH), lambda b, m: (m[b], 0, 0)),
            pl.BlockSpec((1, _FH, _H), lambda b, m: (m[b], 0, 0)),
            pl.BlockSpec((1, 1, _H), lambda b, m: (m[b], 0, 0)),
        ],
        out_specs=pl.BlockSpec((_B, _H), lambda b, m: (b, 0)),
        scratch_shapes=[pltpu.VMEM((_B, _FH), jnp.float32)],
    )
    ys1 = pl.pallas_call(
        _ffn_h1_body,
        grid_spec=gs1,
        out_shape=jax.ShapeDtypeStruct((_P, _H), jnp.float32),
    )(meta, xs, W1, b1.reshape(_E, 1, _F), W2, b2.reshape(_E, 1, _H))

    gs2 = pltpu.PrefetchScalarGridSpec(
        num_scalar_prefetch=1,
        grid=(_NB,),
        in_specs=[
            pl.BlockSpec((_B, _H), lambda b, m: (b, 0)),
            pl.BlockSpec((1, _H, _FH), lambda b, m: (m[b], 0, 1)),
            pl.BlockSpec((1, 1, _FH), lambda b, m: (m[b], 0, 1)),
            pl.BlockSpec((1, _FH, _H), lambda b, m: (m[b], 1, 0)),
            pl.BlockSpec((_B, _H), lambda b, m: (b, 0)),
        ],
        out_specs=pl.BlockSpec((_B, _H), lambda b, m: (b, 0)),
        scratch_shapes=[pltpu.VMEM((_B, _FH), jnp.float32)],
    )
    return pl.pallas_call(
        _ffn_h2_body,
        grid_spec=gs2,
        out_shape=jax.ShapeDtypeStruct((_P, _H), jnp.float32),
    )(meta, xs, W1, b1.reshape(_E, 1, _F), W2, ys1)


@jax.jit
def kernel(x, Wg, bg, W1, b1, W2, b2):
    idx, w = _gating(x, Wg, bg)
    eflat = idx.reshape(-1)

    counts = _count_kernel(eflat)
    if False:
        cnt = counts.reshape(_NW, 16)[:, :_E].sum(axis=0)
        padded = ((cnt + _B - 1) // _B) * _B
        starts = jnp.concatenate([jnp.zeros((1,), jnp.int32),
                                  jnp.cumsum(padded).astype(jnp.int32)])
        cexcl = jnp.concatenate([jnp.zeros((1,), jnp.int32),
                                 jnp.cumsum(cnt).astype(jnp.int32)])[:_E]
        order = jnp.argsort(eflat, stable=True).astype(jnp.int32)
        seg = eflat[order]
        dest_sorted = starts[seg] + (jnp.arange(_A, dtype=jnp.int32) - cexcl[seg])
        dest = jnp.zeros((_A,), jnp.int32).at[order].set(dest_sorted)
        src = jnp.zeros((_P,), jnp.int32).at[dest_sorted].set(order // _K)
        nvalid = starts[_E] // _B
        block_expert = jnp.minimum(
            jnp.searchsorted(starts[1:], jnp.arange(_NB, dtype=jnp.int32) * _B,
                             side='right'), _E - 1).astype(jnp.int32)
        meta = jnp.concatenate([block_expert, nvalid[None].astype(jnp.int32)])
        xs = x[src]
    else:
        xs, dest, meta = _dispatch_kernel(eflat, x, counts)

    ys = _grouped_ffn(meta, xs, W1, b1, W2, b2)

    wflat = w.reshape(-1)
    gathered = ys[dest]
    out = (gathered * wflat[:, None]).reshape(_T, _K, _H).sum(axis=1)
    return out
